# initial kernel scaffold (unmeasured)
import jax
import jax.numpy as jnp
from jax import lax
from jax.experimental import pallas as pl
from jax.experimental.pallas import tpu as pltpu

N_Y = 2
T_LOC = 1024
D = 1024
F = 4096
E_LOC = 8
E = N_Y * E_LOC
C = 384
FB = 4
F_BLK = F // FB


def _exchange(x, router_t):

    def body(x_ref, r_ref, xfull_ref, rfull_ref, send_sems, recv_sems):
        my_x = lax.axis_index("x")
        my_y = lax.axis_index("y")
        nbr = (my_x, 1 - my_y)

        barrier_sem = pltpu.get_barrier_semaphore()
        pl.semaphore_signal(
            barrier_sem, inc=1, device_id=nbr,
            device_id_type=pl.DeviceIdType.MESH,
        )
        pl.semaphore_wait(barrier_sem, 1)

        rdma_x = pltpu.make_async_remote_copy(
            src_ref=x_ref,
            dst_ref=xfull_ref.at[my_y],
            send_sem=send_sems.at[0],
            recv_sem=recv_sems.at[0],
            device_id=nbr,
            device_id_type=pl.DeviceIdType.MESH,
        )
        rdma_x.start()
        rdma_r = pltpu.make_async_remote_copy(
            src_ref=r_ref,
            dst_ref=rfull_ref.at[my_y],
            send_sem=send_sems.at[1],
            recv_sem=recv_sems.at[1],
            device_id=nbr,
            device_id_type=pl.DeviceIdType.MESH,
        )
        rdma_r.start()

        xfull_ref[pl.ds(my_y, 1)] = x_ref[...][None]
        rfull_ref[pl.ds(my_y, 1)] = r_ref[...][None]

        rdma_x.wait()
        rdma_r.wait()

    return pl.pallas_call(
        body,
        out_shape=(
            jax.ShapeDtypeStruct((N_Y, T_LOC, D), jnp.float32),
            jax.ShapeDtypeStruct((N_Y, E_LOC, D), jnp.float32),
        ),
        in_specs=[
            pl.BlockSpec(memory_space=pltpu.VMEM),
            pl.BlockSpec(memory_space=pltpu.VMEM),
        ],
        out_specs=(
            pl.BlockSpec(memory_space=pltpu.VMEM),
            pl.BlockSpec(memory_space=pltpu.VMEM),
        ),
        scratch_shapes=[
            pltpu.SemaphoreType.DMA((2,)),
            pltpu.SemaphoreType.DMA((2,)),
        ],
        compiler_params=pltpu.CompilerParams(collective_id=0),
    )(x, router_t)


def _ffn(xg, W1, W2):

    def body(xg_ref, w1_ref, w2_ref, out_ref, acc_ref):
        fb = pl.program_id(1)
        h = jnp.dot(
            xg_ref[0],
            w1_ref[0].astype(jnp.bfloat16),
            preferred_element_type=jnp.float32,
        )
        h = jnp.maximum(h, 0.0).astype(jnp.bfloat16)
        part = jnp.dot(
            h,
            w2_ref[0].astype(jnp.bfloat16),
            preferred_element_type=jnp.float32,
        )

        @pl.when(fb == 0)
        def _():
            acc_ref[...] = part

        @pl.when(fb != 0)
        def _():
            acc_ref[...] += part

        @pl.when(fb == FB - 1)
        def _():
            out_ref[0] = acc_ref[...]

    return pl.pallas_call(
        body,
        grid=(E_LOC, FB),
        out_shape=jax.ShapeDtypeStruct((E_LOC, C, D), jnp.float32),
        in_specs=[
            pl.BlockSpec((1, C, D), lambda e, fb: (e, 0, 0)),
            pl.BlockSpec((1, D, F_BLK), lambda e, fb: (e, 0, fb)),
            pl.BlockSpec((1, F_BLK, D), lambda e, fb: (e, fb, 0)),
        ],
        out_specs=pl.BlockSpec((1, C, D), lambda e, fb: (e, 0, 0)),
        scratch_shapes=[pltpu.VMEM((C, D), jnp.float32)],
    )(xg, W1, W2)


def _combine(partial):

    def body(p_ref, out_ref, sendbuf, recvbuf, send_sem, recv_sem):
        my_x = lax.axis_index("x")
        my_y = lax.axis_index("y")
        nbr = (my_x, 1 - my_y)

        barrier_sem = pltpu.get_barrier_semaphore()
        pl.semaphore_signal(
            barrier_sem, inc=1, device_id=nbr,
            device_id_type=pl.DeviceIdType.MESH,
        )
        pl.semaphore_wait(barrier_sem, 1)

        sendbuf[...] = p_ref[pl.ds((1 - my_y) * T_LOC, T_LOC), :].astype(
            jnp.bfloat16
        )
        rdma = pltpu.make_async_remote_copy(
            src_ref=sendbuf,
            dst_ref=recvbuf,
            send_sem=send_sem,
            recv_sem=recv_sem,
            device_id=nbr,
            device_id_type=pl.DeviceIdType.MESH,
        )
        rdma.start()
        rdma.wait()
        out_ref[...] = p_ref[pl.ds(my_y * T_LOC, T_LOC), :] + recvbuf[
            ...
        ].astype(jnp.float32)

    return pl.pallas_call(
        body,
        out_shape=jax.ShapeDtypeStruct((T_LOC, D), jnp.float32),
        in_specs=[pl.BlockSpec(memory_space=pltpu.VMEM)],
        out_specs=pl.BlockSpec(memory_space=pltpu.VMEM),
        scratch_shapes=[
            pltpu.VMEM((T_LOC, D), jnp.bfloat16),
            pltpu.VMEM((T_LOC, D), jnp.bfloat16),
            pltpu.SemaphoreType.DMA,
            pltpu.SemaphoreType.DMA,
        ],
        compiler_params=pltpu.CompilerParams(collective_id=1),
    )(partial)


def kernel(x, router, W1, W2):
    my_y = lax.axis_index("y")

    xfull, rfull = _exchange(x, router.T)
    X_full = xfull.reshape(N_Y * T_LOC, D)
    R_full = rfull.reshape(E, D).T

    gates = X_full @ R_full
    top_v, top_i = lax.top_k(gates, 2)
    w = jax.nn.softmax(top_v, axis=-1)

    T = N_Y * T_LOC
    e_flat = top_i.reshape(-1)
    t_flat = jnp.broadcast_to(
        jnp.arange(T, dtype=jnp.int32)[:, None], (T, 2)
    ).reshape(-1)
    w_flat = w.reshape(-1)

    le = e_flat - my_y * E_LOC
    oh = le[:, None] == jnp.arange(E_LOC, dtype=le.dtype)[None, :]
    pos = jnp.cumsum(oh.astype(jnp.int32), axis=0) - 1
    slot = jnp.sum(jnp.where(oh, pos, 0), axis=1)
    valid = (le >= 0) & (le < E_LOC) & (slot < C)
    de = jnp.where(valid, le, E_LOC)
    dslot = jnp.where(valid, slot, 0)

    token_buf = (
        jnp.zeros((E_LOC + 1, C), jnp.int32).at[de, dslot].set(t_flat)
    )[:E_LOC]
    wgt_buf = (
        jnp.zeros((E_LOC + 1, C), jnp.float32).at[de, dslot].set(w_flat)
    )[:E_LOC]

    X_bf = X_full.astype(jnp.bfloat16)
    xg = X_bf[token_buf.reshape(-1)].reshape(E_LOC, C, D)

    yg = _ffn(xg, W1, W2)
    yg = yg * wgt_buf[:, :, None]

    partial = (
        jnp.zeros((T, D), jnp.float32)
        .at[token_buf.reshape(-1)]
        .add(yg.reshape(-1, D))
    )
    return _combine(partial)


# baseline (device time: 264063 ns/iter reference)
import jax
import jax.numpy as jnp
from jax import lax
from jax.experimental import pallas as pl
from jax.experimental.pallas import tpu as pltpu

N_Y = 2
T_LOC = 1024
D = 1024
F = 4096
E_LOC = 8
E = N_Y * E_LOC
C = 384
FB = 4
F_BLK = F // FB


def _exchange(x, router_t):

    def body(x_ref, r_ref, xfull_ref, rfull_ref, send_sems, recv_sems):
        my_x = lax.axis_index("x")
        my_y = lax.axis_index("y")
        nbr = (my_x, 1 - my_y)

        barrier_sem = pltpu.get_barrier_semaphore()
        pl.semaphore_signal(
            barrier_sem, inc=1, device_id=nbr,
            device_id_type=pl.DeviceIdType.MESH,
        )
        pl.semaphore_wait(barrier_sem, 1)

        rdma_x = pltpu.make_async_remote_copy(
            src_ref=x_ref,
            dst_ref=xfull_ref.at[my_y],
            send_sem=send_sems.at[0],
            recv_sem=recv_sems.at[0],
            device_id=nbr,
            device_id_type=pl.DeviceIdType.MESH,
        )
        rdma_x.start()
        rdma_r = pltpu.make_async_remote_copy(
            src_ref=r_ref,
            dst_ref=rfull_ref.at[my_y],
            send_sem=send_sems.at[1],
            recv_sem=recv_sems.at[1],
            device_id=nbr,
            device_id_type=pl.DeviceIdType.MESH,
        )
        rdma_r.start()

        xfull_ref[pl.ds(my_y, 1)] = x_ref[...][None]
        rfull_ref[pl.ds(my_y, 1)] = r_ref[...][None]

        rdma_x.wait()
        rdma_r.wait()

    return pl.pallas_call(
        body,
        out_shape=(
            jax.ShapeDtypeStruct((N_Y, T_LOC, D), jnp.float32),
            jax.ShapeDtypeStruct((N_Y, E_LOC, D), jnp.float32),
        ),
        in_specs=[
            pl.BlockSpec(memory_space=pltpu.VMEM),
            pl.BlockSpec(memory_space=pltpu.VMEM),
        ],
        out_specs=(
            pl.BlockSpec(memory_space=pltpu.VMEM),
            pl.BlockSpec(memory_space=pltpu.VMEM),
        ),
        scratch_shapes=[
            pltpu.SemaphoreType.DMA((2,)),
            pltpu.SemaphoreType.DMA((2,)),
        ],
        compiler_params=pltpu.CompilerParams(collective_id=0),
    )(x, router_t)


def _ffn(xg, W1, W2):

    def body(xg_ref, w1_ref, w2_ref, out_ref, acc_ref):
        fb = pl.program_id(1)
        h = jnp.dot(
            xg_ref[0],
            w1_ref[0].astype(jnp.bfloat16),
            preferred_element_type=jnp.float32,
        )
        h = jnp.maximum(h, 0.0).astype(jnp.bfloat16)
        part = jnp.dot(
            h,
            w2_ref[0].astype(jnp.bfloat16),
            preferred_element_type=jnp.float32,
        )

        @pl.when(fb == 0)
        def _():
            acc_ref[...] = part

        @pl.when(fb != 0)
        def _():
            acc_ref[...] += part

        @pl.when(fb == FB - 1)
        def _():
            out_ref[0] = acc_ref[...]

    return pl.pallas_call(
        body,
        grid=(E_LOC, FB),
        out_shape=jax.ShapeDtypeStruct((E_LOC, C, D), jnp.float32),
        in_specs=[
            pl.BlockSpec((1, C, D), lambda e, fb: (e, 0, 0)),
            pl.BlockSpec((1, D, F_BLK), lambda e, fb: (e, 0, fb)),
            pl.BlockSpec((1, F_BLK, D), lambda e, fb: (e, fb, 0)),
        ],
        out_specs=pl.BlockSpec((1, C, D), lambda e, fb: (e, 0, 0)),
        scratch_shapes=[pltpu.VMEM((C, D), jnp.float32)],
    )(xg, W1, W2)


def _combine(partial):

    def body(p_ref, out_ref, sendbuf, recvbuf, send_sem, recv_sem):
        my_x = lax.axis_index("x")
        my_y = lax.axis_index("y")
        nbr = (my_x, 1 - my_y)

        barrier_sem = pltpu.get_barrier_semaphore()
        pl.semaphore_signal(
            barrier_sem, inc=1, device_id=nbr,
            device_id_type=pl.DeviceIdType.MESH,
        )
        pl.semaphore_wait(barrier_sem, 1)

        sendbuf[...] = p_ref[pl.ds((1 - my_y) * T_LOC, T_LOC), :].astype(
            jnp.bfloat16
        )
        rdma = pltpu.make_async_remote_copy(
            src_ref=sendbuf,
            dst_ref=recvbuf,
            send_sem=send_sem,
            recv_sem=recv_sem,
            device_id=nbr,
            device_id_type=pl.DeviceIdType.MESH,
        )
        rdma.start()
        rdma.wait()
        out_ref[...] = p_ref[pl.ds(my_y * T_LOC, T_LOC), :] + recvbuf[
            ...
        ].astype(jnp.float32)

    return pl.pallas_call(
        body,
        out_shape=jax.ShapeDtypeStruct((T_LOC, D), jnp.float32),
        in_specs=[pl.BlockSpec(memory_space=pltpu.VMEM)],
        out_specs=pl.BlockSpec(memory_space=pltpu.VMEM),
        scratch_shapes=[
            pltpu.VMEM((T_LOC, D), jnp.bfloat16),
            pltpu.VMEM((T_LOC, D), jnp.bfloat16),
            pltpu.SemaphoreType.DMA,
            pltpu.SemaphoreType.DMA,
        ],
        compiler_params=pltpu.CompilerParams(collective_id=1),
    )(partial)


def kernel(x, router, W1, W2):
    my_y = lax.axis_index("y")

    xfull, rfull = _exchange(x, router.T)
    X_full = xfull.reshape(N_Y * T_LOC, D)
    R_full = rfull.reshape(E, D).T

    gates = jnp.dot(
        X_full, R_full, precision=lax.Precision.HIGHEST
    )
    top_v, top_i = lax.top_k(gates, 2)
    w = jax.nn.softmax(top_v, axis=-1)

    T = N_Y * T_LOC
    e_flat = top_i.reshape(-1)
    t_flat = jnp.broadcast_to(
        jnp.arange(T, dtype=jnp.int32)[:, None], (T, 2)
    ).reshape(-1)
    w_flat = w.reshape(-1)

    le = e_flat - my_y * E_LOC
    oh = le[:, None] == jnp.arange(E_LOC, dtype=le.dtype)[None, :]
    pos = jnp.cumsum(oh.astype(jnp.int32), axis=0) - 1
    slot = jnp.sum(jnp.where(oh, pos, 0), axis=1)
    valid = (le >= 0) & (le < E_LOC) & (slot < C)
    de = jnp.where(valid, le, E_LOC)
    dslot = jnp.where(valid, slot, 0)

    token_buf = (
        jnp.zeros((E_LOC + 1, C), jnp.int32).at[de, dslot].set(t_flat)
    )[:E_LOC]
    wgt_buf = (
        jnp.zeros((E_LOC + 1, C), jnp.float32).at[de, dslot].set(w_flat)
    )[:E_LOC]

    X_bf = X_full.astype(jnp.bfloat16)
    xg = X_bf[token_buf.reshape(-1)].reshape(E_LOC, C, D)

    yg = _ffn(xg, W1, W2)
    yg = yg * wgt_buf[:, :, None]

    partial = (
        jnp.zeros((T, D), jnp.float32)
        .at[token_buf.reshape(-1)]
        .add(yg.reshape(-1, D))
    )
    return _combine(partial)
